# Initial kernel scaffold; baseline (speedup 1.0000x reference)
import jax
import jax.numpy as jnp
from jax.experimental import pallas as pl


def kernel(positions, velocities, noisy_displacements, noisy_delta_v, timesteps, params):
    raise NotImplementedError("write your pallas kernel here")


# speedup vs baseline: 3.4777x; 3.4777x over previous
import functools

import jax
import jax.numpy as jnp
import numpy as np
from jax import lax
from jax.experimental import pallas as pl

_H = 128
_K = 32
_TD = 64
_PADPOS = 1.0e6

_BLKG = 256
_BLKE = 128
_TW = 256


def _silu(x):
    return x * jax.nn.sigmoid(x)


def _pre0_body(feats_ref, pos_ref, wint, binp, nf_out, tab_out):
    feats = feats_ref[...]
    nf = jnp.dot(feats, wint[...], preferred_element_type=jnp.float32) + binp[...]
    nf_out[...] = nf
    tab_out[...] = jnp.concatenate(
        [nf, pos_ref[...],
         jnp.zeros((nf.shape[0], _TW - _H - 16), jnp.float32)], axis=1)


def _gb_body(pos_ref, post_ref, idx_out, *, n, blk):
    rows = pos_ref[...]
    cols = post_ref[...]
    d2 = (cols[0, 0:1, :] - rows[:, 0:1]) ** 2
    d2 = d2 + (cols[0, 1:2, :] - rows[:, 1:2]) ** 2
    d2 = d2 + (cols[0, 2:3, :] - rows[:, 2:3]) ** 2
    b = pl.program_id(0)
    i = pl.program_id(1)
    dist = jnp.sqrt(d2 + 1e-12)
    col_iota = lax.broadcasted_iota(jnp.int32, (blk, n), 1)
    row_ids = i * blk + lax.broadcasted_iota(jnp.int32, (blk, n), 0)
    valid = (dist < 6.0) & (col_iota != row_ids)
    dm = jnp.where(valid, dist, jnp.inf)
    pad = jnp.int32(2 * n)
    big = jnp.int32(n)
    off = b * n
    slots = []
    for _ in range(_K):
        mn = jnp.min(dm, axis=1, keepdims=True)
        amin = jnp.min(jnp.where(dm == mn, col_iota, big), axis=1,
                       keepdims=True)
        dm = jnp.where(col_iota == amin, jnp.inf, dm)
        slots.append(jnp.where(jnp.isinf(mn), pad, amin + off))
    idx_out[0] = jnp.concatenate(slots, axis=1)


def _en_body(gath_ref, pos_ref, nf_ref,
             w1p, b1, w2t, b2, c1t, bc1, c2t, bc2v,
             n1t, nb1, n2t, nb2,
             nf_out, pos_out, tab_out):
    g = gath_ref[...]
    nfj = g[:, :, :_H]
    pj = g[:, :, _H:_H + 16]
    pi = pos_ref[...]
    nfi = nf_ref[...]
    diff = pj - pi[:, None, :]
    radial = (diff[:, :, 0:1] ** 2 + diff[:, :, 1:2] ** 2
              + diff[:, :, 2:3] ** 2)
    em = (jnp.sqrt(radial + 1e-12) < 6.0).astype(jnp.float32)
    src = jnp.broadcast_to(nfi[:, None, :], (_BLKE, _K, _H))
    ei = jnp.concatenate(
        [src, nfj, radial,
         jnp.zeros((_BLKE, _K, 384 - 2 * _H - 1), jnp.float32)], axis=2)
    ei2 = ei.reshape(_BLKE * _K, 384)
    pre = jnp.dot(ei2, w1p[...], preferred_element_type=jnp.float32) + b1[...]
    m2 = _silu(jnp.dot(_silu(pre), w2t[...],
                       preferred_element_type=jnp.float32) + b2[...])
    m3 = m2.reshape(_BLKE, _K, _H) * em
    mflat = m3.reshape(_BLKE * _K, _H)
    c1 = _silu(jnp.dot(mflat, c1t[...], preferred_element_type=jnp.float32)
               + bc1[...])
    cs_flat = jnp.dot(c1, c2t[...], preferred_element_type=jnp.float32)
    cs = (cs_flat[:, 0:1] + bc2v[...][0:1, 0:1]).reshape(_BLKE, _K, 1)
    wq = jnp.sum((diff * cs) * em, axis=1)
    posn = pi + wq
    agg = jnp.sum(m3, axis=1)
    cat = jnp.concatenate([nfi, agg], axis=1)
    na = _silu(jnp.dot(cat, n1t[...], preferred_element_type=jnp.float32)
               + nb1[...])
    nfn = nfi + (jnp.dot(na, n2t[...], preferred_element_type=jnp.float32)
                 + nb2[...])
    nf_out[...] = nfn
    pos_out[...] = posn
    tab_out[...] = jnp.concatenate(
        [nfn, posn, jnp.zeros((nfn.shape[0], _TW - _H - 16), jnp.float32)],
        axis=1)


def _full(shape):
    return pl.BlockSpec(shape, lambda *_: tuple(0 for _ in shape))


def _pad_lanes(x, w):
    return jnp.pad(x, ((0, 0), (0, w - x.shape[1])))


def kernel(positions, velocities, noisy_displacements, noisy_delta_v,
           timesteps, params):
    B, N, _ = positions.shape
    M = B * N
    f32 = jnp.float32

    half = _TD // 2
    freqs = jnp.exp(-np.log(10000.0) * jnp.arange(half, dtype=f32) / half)
    args = timesteps.astype(f32)[:, None] * freqs[None, :]
    tf = jnp.concatenate([jnp.sin(args), jnp.cos(args)], axis=-1)
    v, dx, dv = velocities, noisy_displacements, noisy_delta_v
    nf0 = jnp.concatenate([
        jnp.linalg.norm(v, axis=-1, keepdims=True),
        jnp.linalg.norm(dx, axis=-1, keepdims=True),
        jnp.linalg.norm(dv, axis=-1, keepdims=True),
        (v * dx).sum(-1, keepdims=True),
        (v * dv).sum(-1, keepdims=True),
        (dx * dv).sum(-1, keepdims=True)], axis=-1)
    feats = jnp.concatenate(
        [nf0, jnp.broadcast_to(tf[:, None, :], (B, N, _TD))], axis=-1)
    feats = _pad_lanes(feats.reshape(M, 6 + _TD), _H)
    pos16 = _pad_lanes(positions.reshape(M, 3), 16)

    def lt(p):
        return p[0].T

    wint = _pad_lanes(jnp.pad(lt(params['input_proj']),
                              ((0, _H - (6 + _TD)), (0, 0))), _H)
    binp = params['input_proj'][1][None, :]
    lw = []
    for lp in params['layers']:
        w1 = lp['edge1'][0]
        lw.append(dict(
            w1p=jnp.pad(w1.T, ((0, 384 - (2 * _H + 1)), (0, 0))),
            b1=lp['edge1'][1][None, :],
            w2t=lt(lp['edge2']), b2=lp['edge2'][1][None, :],
            c1t=lt(lp['coord1']), bc1=lp['coord1'][1][None, :],
            c2t=jnp.zeros((_H, _H), f32).at[:, 0].set(lp['coord2'][0][0]),
            bc2v=jnp.broadcast_to(lp['coord2'][1][0], (1, _H)),
            n1t=lp['node1'][0].T, nb1=lp['node1'][1][None, :],
            n2t=lt(lp['node2']), nb2=lp['node2'][1][None, :]))

    nblk = M // _BLKE
    grid_rows = pl.BlockSpec((_BLKE, _H), lambda i: (i, 0))
    grid_r16 = pl.BlockSpec((_BLKE, 16), lambda i: (i, 0))
    grid_rtw = pl.BlockSpec((_BLKE, _TW), lambda i: (i, 0))
    grid_g = pl.BlockSpec((_BLKE, _K, _TW), lambda i: (i, 0, 0))

    nf, table = pl.pallas_call(
        _pre0_body,
        grid=(nblk,),
        in_specs=[grid_rows, grid_r16, _full((_H, _H)), _full((1, _H))],
        out_specs=[grid_rows, grid_rtw],
        out_shape=[jax.ShapeDtypeStruct((M, _H), f32),
                   jax.ShapeDtypeStruct((M, _TW), f32)],
    )(feats, pos16, wint, binp)

    padrows = jnp.zeros((8, _TW), f32).at[:, _H].set(_PADPOS)

    for L in range(3):
        posT = pos16[:, :3].reshape(B, N, 3).transpose(0, 2, 1)
        idxg = pl.pallas_call(
            functools.partial(_gb_body, n=N, blk=_BLKG),
            grid=(B, N // _BLKG),
            in_specs=[pl.BlockSpec((_BLKG, 16),
                                   lambda b, i: (b * (N // _BLKG) + i, 0)),
                      pl.BlockSpec((1, 3, N), lambda b, i: (b, 0, 0))],
            out_specs=pl.BlockSpec((1, _BLKG, _K), lambda b, i: (b, i, 0)),
            out_shape=jax.ShapeDtypeStruct((B, N, _K), jnp.int32),
        )(pos16, posT)

        tablep = jnp.concatenate([table, padrows], axis=0)
        gath = tablep[idxg.reshape(M * _K)].reshape(M, _K, _TW)

        w = lw[L]
        nf, pos16, table = pl.pallas_call(
            _en_body,
            grid=(nblk,),
            in_specs=[grid_g, grid_r16, grid_rows,
                      _full((384, _H)), _full((1, _H)),
                      _full((_H, _H)), _full((1, _H)),
                      _full((_H, _H)), _full((1, _H)),
                      _full((_H, _H)), _full((1, _H)),
                      _full((2 * _H, _H)), _full((1, _H)),
                      _full((_H, _H)), _full((1, _H))],
            out_specs=[grid_rows, grid_r16, grid_rtw],
            out_shape=[jax.ShapeDtypeStruct((M, _H), f32),
                       jax.ShapeDtypeStruct((M, 16), f32),
                       jax.ShapeDtypeStruct((M, _TW), f32)],
        )(gath, pos16, nf,
          w['w1p'], w['b1'], w['w2t'], w['b2'], w['c1t'], w['bc1'],
          w['c2t'], w['bc2v'], w['n1t'], w['nb1'], w['n2t'], w['nb2'])

    nfB = nf.reshape(B, N, _H)
    vc = nfB @ params['vh1'][0].T + params['vh1'][1]
    vc = jax.nn.silu(vc) @ params['vh2'][0].T + params['vh2'][1]
    cx, cv = vc[..., :3], vc[..., 3:]
    basis = jnp.stack([dx, dv, v], axis=-2)
    eps_x = (cx[..., None] * basis).sum(axis=-2)
    eps_v = (cv[..., None] * basis).sum(axis=-2)
    return eps_x, eps_v


# speedup vs baseline: 4.5398x; 1.3054x over previous
import functools

import jax
import jax.numpy as jnp
import numpy as np
from jax import lax
from jax.experimental import pallas as pl
from jax.experimental.pallas import tpu as pltpu
from jax.experimental.pallas import tpu_sc as plsc

_H = 128
_K = 32
_TD = 64
_PADPOS = 1.0e6

_BLKG = 256
_BLKE = 128
_TW = 256
_NW = 32
_GCH = 128


def _silu(x):
    return x * jax.nn.sigmoid(x)


def _pre0_body(feats_ref, pos_ref, wint, binp, nf_out, tab_out):
    feats = feats_ref[...]
    nf = jnp.dot(feats, wint[...], preferred_element_type=jnp.float32) + binp[...]
    nf_out[...] = nf
    tab_out[...] = jnp.concatenate(
        [nf, pos_ref[...],
         jnp.zeros((nf.shape[0], _TW - _H - 16), jnp.float32)], axis=1)


def _gb_body(pos_ref, post_ref, idx_out, *, n, blk):
    rows = pos_ref[...]
    cols = post_ref[...]
    d2 = (cols[0, 0:1, :] - rows[:, 0:1]) ** 2
    d2 = d2 + (cols[0, 1:2, :] - rows[:, 1:2]) ** 2
    d2 = d2 + (cols[0, 2:3, :] - rows[:, 2:3]) ** 2
    b = pl.program_id(0)
    i = pl.program_id(1)
    dist = jnp.sqrt(d2 + 1e-12)
    col_iota = lax.broadcasted_iota(jnp.int32, (blk, n), 1)
    row_ids = i * blk + lax.broadcasted_iota(jnp.int32, (blk, n), 0)
    valid = (dist < 6.0) & (col_iota != row_ids)
    dm = jnp.where(valid, dist, jnp.inf)
    pad = jnp.int32(2 * n)
    big = jnp.int32(n)
    off = b * n
    slots = []
    for _ in range(_K):
        mn = jnp.min(dm, axis=1, keepdims=True)
        amin = jnp.min(jnp.where(dm == mn, col_iota, big), axis=1,
                       keepdims=True)
        dm = jnp.where(col_iota == amin, jnp.inf, dm)
        slots.append(jnp.where(jnp.isinf(mn), pad, amin + off))
    idx_out[0] = jnp.concatenate(slots, axis=1)


def _sc_gather(table, idx):
    e = idx.shape[0]
    d = table.shape[1]
    per_w = e // _NW
    nch = per_w // _GCH
    idx3 = idx.reshape(_NW, nch, _GCH)
    mesh = plsc.VectorSubcoreMesh(core_axis_name="c", subcore_axis_name="s")

    @functools.partial(
        pl.kernel,
        out_type=jax.ShapeDtypeStruct((e, d), jnp.float32),
        mesh=mesh,
        scratch_types=[
            pltpu.VMEM((nch, _GCH), jnp.int32),
            pltpu.VMEM((_GCH, d), jnp.float32),
            pltpu.SemaphoreType.DMA,
        ],
    )
    def gk(table_hbm, idx_hbm, out_hbm, idx_v, buf, sem):
        wid = lax.axis_index("s") * 2 + lax.axis_index("c")
        base = wid * per_w
        pltpu.sync_copy(idx_hbm.at[wid], idx_v)

        def body(c, carry):
            pltpu.async_copy(table_hbm.at[idx_v.at[c]], buf, sem).wait()
            pltpu.sync_copy(buf, out_hbm.at[pl.ds(base + c * _GCH, _GCH)])
            return carry

        lax.fori_loop(0, nch, body, 0)

    return gk(table, idx3)


def _en_body(gath_ref, pos_ref, nf_ref,
             w1p, b1, w2t, b2, c1t, bc1, c2t, bc2v,
             n1t, nb1, n2t, nb2,
             nf_out, pos_out, tab_out):
    g = gath_ref[...]
    nfj = g[:, :, :_H]
    pj = g[:, :, _H:_H + 16]
    pi = pos_ref[...]
    nfi = nf_ref[...]
    diff = pj - pi[:, None, :]
    radial = (diff[:, :, 0:1] ** 2 + diff[:, :, 1:2] ** 2
              + diff[:, :, 2:3] ** 2)
    em = (jnp.sqrt(radial + 1e-12) < 6.0).astype(jnp.float32)
    src = jnp.broadcast_to(nfi[:, None, :], (_BLKE, _K, _H))
    ei = jnp.concatenate(
        [src, nfj, radial,
         jnp.zeros((_BLKE, _K, 384 - 2 * _H - 1), jnp.float32)], axis=2)
    ei2 = ei.reshape(_BLKE * _K, 384)
    pre = jnp.dot(ei2, w1p[...], preferred_element_type=jnp.float32) + b1[...]
    m2 = _silu(jnp.dot(_silu(pre), w2t[...],
                       preferred_element_type=jnp.float32) + b2[...])
    m3 = m2.reshape(_BLKE, _K, _H) * em
    mflat = m3.reshape(_BLKE * _K, _H)
    c1 = _silu(jnp.dot(mflat, c1t[...], preferred_element_type=jnp.float32)
               + bc1[...])
    cs_flat = jnp.dot(c1, c2t[...], preferred_element_type=jnp.float32)
    cs = (cs_flat[:, 0:1] + bc2v[...][0:1, 0:1]).reshape(_BLKE, _K, 1)
    wq = jnp.sum((diff * cs) * em, axis=1)
    posn = pi + wq
    agg = jnp.sum(m3, axis=1)
    cat = jnp.concatenate([nfi, agg], axis=1)
    na = _silu(jnp.dot(cat, n1t[...], preferred_element_type=jnp.float32)
               + nb1[...])
    nfn = nfi + (jnp.dot(na, n2t[...], preferred_element_type=jnp.float32)
                 + nb2[...])
    nf_out[...] = nfn
    pos_out[...] = posn
    tab_out[...] = jnp.concatenate(
        [nfn, posn, jnp.zeros((nfn.shape[0], _TW - _H - 16), jnp.float32)],
        axis=1)


def _full(shape):
    return pl.BlockSpec(shape, lambda *_: tuple(0 for _ in shape))


def _pad_lanes(x, w):
    return jnp.pad(x, ((0, 0), (0, w - x.shape[1])))


def kernel(positions, velocities, noisy_displacements, noisy_delta_v,
           timesteps, params):
    B, N, _ = positions.shape
    M = B * N
    f32 = jnp.float32

    half = _TD // 2
    freqs = jnp.exp(-np.log(10000.0) * jnp.arange(half, dtype=f32) / half)
    args = timesteps.astype(f32)[:, None] * freqs[None, :]
    tf = jnp.concatenate([jnp.sin(args), jnp.cos(args)], axis=-1)
    v, dx, dv = velocities, noisy_displacements, noisy_delta_v
    nf0 = jnp.concatenate([
        jnp.linalg.norm(v, axis=-1, keepdims=True),
        jnp.linalg.norm(dx, axis=-1, keepdims=True),
        jnp.linalg.norm(dv, axis=-1, keepdims=True),
        (v * dx).sum(-1, keepdims=True),
        (v * dv).sum(-1, keepdims=True),
        (dx * dv).sum(-1, keepdims=True)], axis=-1)
    feats = jnp.concatenate(
        [nf0, jnp.broadcast_to(tf[:, None, :], (B, N, _TD))], axis=-1)
    feats = _pad_lanes(feats.reshape(M, 6 + _TD), _H)
    pos16 = _pad_lanes(positions.reshape(M, 3), 16)

    def lt(p):
        return p[0].T

    wint = _pad_lanes(jnp.pad(lt(params['input_proj']),
                              ((0, _H - (6 + _TD)), (0, 0))), _H)
    binp = params['input_proj'][1][None, :]
    lw = []
    for lp in params['layers']:
        w1 = lp['edge1'][0]
        lw.append(dict(
            w1p=jnp.pad(w1.T, ((0, 384 - (2 * _H + 1)), (0, 0))),
            b1=lp['edge1'][1][None, :],
            w2t=lt(lp['edge2']), b2=lp['edge2'][1][None, :],
            c1t=lt(lp['coord1']), bc1=lp['coord1'][1][None, :],
            c2t=jnp.zeros((_H, _H), f32).at[:, 0].set(lp['coord2'][0][0]),
            bc2v=jnp.broadcast_to(lp['coord2'][1][0], (1, _H)),
            n1t=lp['node1'][0].T, nb1=lp['node1'][1][None, :],
            n2t=lt(lp['node2']), nb2=lp['node2'][1][None, :]))

    nblk = M // _BLKE
    grid_rows = pl.BlockSpec((_BLKE, _H), lambda i: (i, 0))
    grid_r16 = pl.BlockSpec((_BLKE, 16), lambda i: (i, 0))
    grid_rtw = pl.BlockSpec((_BLKE, _TW), lambda i: (i, 0))
    grid_g = pl.BlockSpec((_BLKE, _K, _TW), lambda i: (i, 0, 0))

    nf, table = pl.pallas_call(
        _pre0_body,
        grid=(nblk,),
        in_specs=[grid_rows, grid_r16, _full((_H, _H)), _full((1, _H))],
        out_specs=[grid_rows, grid_rtw],
        out_shape=[jax.ShapeDtypeStruct((M, _H), f32),
                   jax.ShapeDtypeStruct((M, _TW), f32)],
    )(feats, pos16, wint, binp)

    padrows = jnp.zeros((8, _TW), f32).at[:, _H].set(_PADPOS)

    for L in range(3):
        posT = pos16[:, :3].reshape(B, N, 3).transpose(0, 2, 1)
        idxg = pl.pallas_call(
            functools.partial(_gb_body, n=N, blk=_BLKG),
            grid=(B, N // _BLKG),
            in_specs=[pl.BlockSpec((_BLKG, 16),
                                   lambda b, i: (b * (N // _BLKG) + i, 0)),
                      pl.BlockSpec((1, 3, N), lambda b, i: (b, 0, 0))],
            out_specs=pl.BlockSpec((1, _BLKG, _K), lambda b, i: (b, i, 0)),
            out_shape=jax.ShapeDtypeStruct((B, N, _K), jnp.int32),
        )(pos16, posT)

        tablep = jnp.concatenate([table, padrows], axis=0)
        gath = _sc_gather(tablep, idxg.reshape(M * _K)).reshape(M, _K, _TW)

        w = lw[L]
        nf, pos16, table = pl.pallas_call(
            _en_body,
            grid=(nblk,),
            in_specs=[grid_g, grid_r16, grid_rows,
                      _full((384, _H)), _full((1, _H)),
                      _full((_H, _H)), _full((1, _H)),
                      _full((_H, _H)), _full((1, _H)),
                      _full((_H, _H)), _full((1, _H)),
                      _full((2 * _H, _H)), _full((1, _H)),
                      _full((_H, _H)), _full((1, _H))],
            out_specs=[grid_rows, grid_r16, grid_rtw],
            out_shape=[jax.ShapeDtypeStruct((M, _H), f32),
                       jax.ShapeDtypeStruct((M, 16), f32),
                       jax.ShapeDtypeStruct((M, _TW), f32)],
        )(gath, pos16, nf,
          w['w1p'], w['b1'], w['w2t'], w['b2'], w['c1t'], w['bc1'],
          w['c2t'], w['bc2v'], w['n1t'], w['nb1'], w['n2t'], w['nb2'])

    nfB = nf.reshape(B, N, _H)
    vc = nfB @ params['vh1'][0].T + params['vh1'][1]
    vc = jax.nn.silu(vc) @ params['vh2'][0].T + params['vh2'][1]
    cx, cv = vc[..., :3], vc[..., 3:]
    basis = jnp.stack([dx, dv, v], axis=-2)
    eps_x = (cx[..., None] * basis).sum(axis=-2)
    eps_v = (cv[..., None] * basis).sum(axis=-2)
    return eps_x, eps_v


# speedup vs baseline: 4.5745x; 1.0076x over previous
import functools

import jax
import jax.numpy as jnp
import numpy as np
from jax import lax
from jax.experimental import pallas as pl
from jax.experimental.pallas import tpu as pltpu
from jax.experimental.pallas import tpu_sc as plsc

_H = 128
_K = 32
_TD = 64
_PADPOS = 1.0e6

_BLKG = 256
_BLKE = 128
_TW = 256
_NW = 32
_GCH = 128


def _silu(x):
    return x * jax.nn.sigmoid(x)


def _pre0_body(feats_ref, pos_ref, wint, binp, nf_out, tab_out):
    feats = feats_ref[...]
    nf = jnp.dot(feats, wint[...], preferred_element_type=jnp.float32) + binp[...]
    nf_out[...] = nf
    tab_out[...] = jnp.concatenate(
        [nf, pos_ref[...],
         jnp.zeros((nf.shape[0], _TW - _H - 16), jnp.float32)], axis=1)


def _gb_body(pos_ref, post_ref, idx_out, *, n, blk):
    rows = pos_ref[...]
    cols = post_ref[...]
    d2 = (cols[0, 0:1, :] - rows[:, 0:1]) ** 2
    d2 = d2 + (cols[0, 1:2, :] - rows[:, 1:2]) ** 2
    d2 = d2 + (cols[0, 2:3, :] - rows[:, 2:3]) ** 2
    b = pl.program_id(0)
    i = pl.program_id(1)
    dist = jnp.sqrt(d2 + 1e-12)
    col_iota = lax.broadcasted_iota(jnp.int32, (blk, n), 1)
    row_ids = i * blk + lax.broadcasted_iota(jnp.int32, (blk, n), 0)
    valid = (dist < 6.0) & (col_iota != row_ids)
    dm = jnp.where(valid, dist, jnp.inf)
    pad = jnp.int32(2 * n)
    big = jnp.int32(n)
    off = b * n
    slots = []
    for _ in range(_K):
        mn = jnp.min(dm, axis=1, keepdims=True)
        amin = jnp.min(jnp.where(dm == mn, col_iota, big), axis=1,
                       keepdims=True)
        dm = jnp.where(col_iota == amin, jnp.inf, dm)
        slots.append(jnp.where(jnp.isinf(mn), pad, amin + off))
    idx_out[0] = jnp.concatenate(slots, axis=1)


def _sc_gather(table, idx):
    e = idx.shape[0]
    d = table.shape[1]
    per_w = e // _NW
    nch = per_w // _GCH
    idx3 = idx.reshape(_NW, nch, _GCH)
    mesh = plsc.VectorSubcoreMesh(core_axis_name="c", subcore_axis_name="s")

    @functools.partial(
        pl.kernel,
        out_type=jax.ShapeDtypeStruct((e, d), jnp.float32),
        mesh=mesh,
        scratch_types=[
            pltpu.VMEM((nch, _GCH), jnp.int32),
            pltpu.VMEM((_GCH, d), jnp.float32),
            pltpu.VMEM((_GCH, d), jnp.float32),
            pltpu.SemaphoreType.DMA,
            pltpu.SemaphoreType.DMA,
        ],
    )
    def gk(table_hbm, idx_hbm, out_hbm, idx_v, buf0, buf1, sem0, sem1):
        wid = lax.axis_index("s") * 2 + lax.axis_index("c")
        base = wid * per_w
        pltpu.sync_copy(idx_hbm.at[wid], idx_v)

        def start(c, buf, sem):
            return pltpu.async_copy(table_hbm.at[idx_v.at[c]], buf, sem)

        bufs = (buf0, buf1)
        sems = (sem0, sem1)
        pending = {0: start(0, buf0, sem0), 1: start(1, buf1, sem1)}
        for c in range(nch):
            buf, sem = bufs[c % 2], sems[c % 2]
            pending.pop(c).wait()
            pltpu.sync_copy(buf, out_hbm.at[pl.ds(base + c * _GCH, _GCH)])
            if c + 2 < nch:
                pending[c + 2] = start(c + 2, buf, sem)

    return gk(table, idx3)


def _en_body(gath_ref, pos_ref, nf_ref,
             w1p, b1, w2t, b2, c1t, bc1, c2t, bc2v,
             n1t, nb1, n2t, nb2,
             nf_out, pos_out, tab_out):
    g = gath_ref[...]
    nfj = g[:, :, :_H]
    pj = g[:, :, _H:_H + 16]
    pi = pos_ref[...]
    nfi = nf_ref[...]
    diff = pj - pi[:, None, :]
    radial = (diff[:, :, 0:1] ** 2 + diff[:, :, 1:2] ** 2
              + diff[:, :, 2:3] ** 2)
    em = (jnp.sqrt(radial + 1e-12) < 6.0).astype(jnp.float32)
    src = jnp.broadcast_to(nfi[:, None, :], (_BLKE, _K, _H))
    ei = jnp.concatenate(
        [src, nfj, radial,
         jnp.zeros((_BLKE, _K, 384 - 2 * _H - 1), jnp.float32)], axis=2)
    ei2 = ei.reshape(_BLKE * _K, 384)
    pre = jnp.dot(ei2, w1p[...], preferred_element_type=jnp.float32) + b1[...]
    m2 = _silu(jnp.dot(_silu(pre), w2t[...],
                       preferred_element_type=jnp.float32) + b2[...])
    m3 = m2.reshape(_BLKE, _K, _H) * em
    mflat = m3.reshape(_BLKE * _K, _H)
    c1 = _silu(jnp.dot(mflat, c1t[...], preferred_element_type=jnp.float32)
               + bc1[...])
    cs_flat = jnp.dot(c1, c2t[...], preferred_element_type=jnp.float32)
    cs = (cs_flat[:, 0:1] + bc2v[...][0:1, 0:1]).reshape(_BLKE, _K, 1)
    wq = jnp.sum((diff * cs) * em, axis=1)
    posn = pi + wq
    agg = jnp.sum(m3, axis=1)
    cat = jnp.concatenate([nfi, agg], axis=1)
    na = _silu(jnp.dot(cat, n1t[...], preferred_element_type=jnp.float32)
               + nb1[...])
    nfn = nfi + (jnp.dot(na, n2t[...], preferred_element_type=jnp.float32)
                 + nb2[...])
    nf_out[...] = nfn
    pos_out[...] = posn
    tab_out[...] = jnp.concatenate(
        [nfn, posn, jnp.zeros((nfn.shape[0], _TW - _H - 16), jnp.float32)],
        axis=1)


def _full(shape):
    return pl.BlockSpec(shape, lambda *_: tuple(0 for _ in shape))


def _pad_lanes(x, w):
    return jnp.pad(x, ((0, 0), (0, w - x.shape[1])))


def kernel(positions, velocities, noisy_displacements, noisy_delta_v,
           timesteps, params):
    B, N, _ = positions.shape
    M = B * N
    f32 = jnp.float32

    half = _TD // 2
    freqs = jnp.exp(-np.log(10000.0) * jnp.arange(half, dtype=f32) / half)
    args = timesteps.astype(f32)[:, None] * freqs[None, :]
    tf = jnp.concatenate([jnp.sin(args), jnp.cos(args)], axis=-1)
    v, dx, dv = velocities, noisy_displacements, noisy_delta_v
    nf0 = jnp.concatenate([
        jnp.linalg.norm(v, axis=-1, keepdims=True),
        jnp.linalg.norm(dx, axis=-1, keepdims=True),
        jnp.linalg.norm(dv, axis=-1, keepdims=True),
        (v * dx).sum(-1, keepdims=True),
        (v * dv).sum(-1, keepdims=True),
        (dx * dv).sum(-1, keepdims=True)], axis=-1)
    feats = jnp.concatenate(
        [nf0, jnp.broadcast_to(tf[:, None, :], (B, N, _TD))], axis=-1)
    feats = _pad_lanes(feats.reshape(M, 6 + _TD), _H)
    pos16 = _pad_lanes(positions.reshape(M, 3), 16)

    def lt(p):
        return p[0].T

    wint = _pad_lanes(jnp.pad(lt(params['input_proj']),
                              ((0, _H - (6 + _TD)), (0, 0))), _H)
    binp = params['input_proj'][1][None, :]
    lw = []
    for lp in params['layers']:
        w1 = lp['edge1'][0]
        lw.append(dict(
            w1p=jnp.pad(w1.T, ((0, 384 - (2 * _H + 1)), (0, 0))),
            b1=lp['edge1'][1][None, :],
            w2t=lt(lp['edge2']), b2=lp['edge2'][1][None, :],
            c1t=lt(lp['coord1']), bc1=lp['coord1'][1][None, :],
            c2t=jnp.zeros((_H, _H), f32).at[:, 0].set(lp['coord2'][0][0]),
            bc2v=jnp.broadcast_to(lp['coord2'][1][0], (1, _H)),
            n1t=lp['node1'][0].T, nb1=lp['node1'][1][None, :],
            n2t=lt(lp['node2']), nb2=lp['node2'][1][None, :]))

    nblk = M // _BLKE
    grid_rows = pl.BlockSpec((_BLKE, _H), lambda i: (i, 0))
    grid_r16 = pl.BlockSpec((_BLKE, 16), lambda i: (i, 0))
    grid_rtw = pl.BlockSpec((_BLKE, _TW), lambda i: (i, 0))
    grid_g = pl.BlockSpec((_BLKE, _K, _TW), lambda i: (i, 0, 0))

    nf, table = pl.pallas_call(
        _pre0_body,
        grid=(nblk,),
        in_specs=[grid_rows, grid_r16, _full((_H, _H)), _full((1, _H))],
        out_specs=[grid_rows, grid_rtw],
        out_shape=[jax.ShapeDtypeStruct((M, _H), f32),
                   jax.ShapeDtypeStruct((M, _TW), f32)],
    )(feats, pos16, wint, binp)

    padrows = jnp.zeros((8, _TW), f32).at[:, _H].set(_PADPOS)

    for L in range(3):
        posT = pos16[:, :3].reshape(B, N, 3).transpose(0, 2, 1)
        idxg = pl.pallas_call(
            functools.partial(_gb_body, n=N, blk=_BLKG),
            grid=(B, N // _BLKG),
            in_specs=[pl.BlockSpec((_BLKG, 16),
                                   lambda b, i: (b * (N // _BLKG) + i, 0)),
                      pl.BlockSpec((1, 3, N), lambda b, i: (b, 0, 0))],
            out_specs=pl.BlockSpec((1, _BLKG, _K), lambda b, i: (b, i, 0)),
            out_shape=jax.ShapeDtypeStruct((B, N, _K), jnp.int32),
        )(pos16, posT)

        tablep = jnp.concatenate([table, padrows], axis=0)
        gath = _sc_gather(tablep, idxg.reshape(M * _K)).reshape(M, _K, _TW)

        w = lw[L]
        nf, pos16, table = pl.pallas_call(
            _en_body,
            grid=(nblk,),
            in_specs=[grid_g, grid_r16, grid_rows,
                      _full((384, _H)), _full((1, _H)),
                      _full((_H, _H)), _full((1, _H)),
                      _full((_H, _H)), _full((1, _H)),
                      _full((_H, _H)), _full((1, _H)),
                      _full((2 * _H, _H)), _full((1, _H)),
                      _full((_H, _H)), _full((1, _H))],
            out_specs=[grid_rows, grid_r16, grid_rtw],
            out_shape=[jax.ShapeDtypeStruct((M, _H), f32),
                       jax.ShapeDtypeStruct((M, 16), f32),
                       jax.ShapeDtypeStruct((M, _TW), f32)],
        )(gath, pos16, nf,
          w['w1p'], w['b1'], w['w2t'], w['b2'], w['c1t'], w['bc1'],
          w['c2t'], w['bc2v'], w['n1t'], w['nb1'], w['n2t'], w['nb2'])

    nfB = nf.reshape(B, N, _H)
    vc = nfB @ params['vh1'][0].T + params['vh1'][1]
    vc = jax.nn.silu(vc) @ params['vh2'][0].T + params['vh2'][1]
    cx, cv = vc[..., :3], vc[..., 3:]
    basis = jnp.stack([dx, dv, v], axis=-2)
    eps_x = (cx[..., None] * basis).sum(axis=-2)
    eps_v = (cv[..., None] * basis).sum(axis=-2)
    return eps_x, eps_v
